# Initial kernel scaffold; baseline (speedup 1.0000x reference)
import jax
import jax.numpy as jnp
from jax.experimental import pallas as pl


def kernel(x, edge_index, e_features, params):
    raise NotImplementedError("write your pallas kernel here")


# speedup vs baseline: 2.9875x; 2.9875x over previous
import functools

import jax
import jax.numpy as jnp
from jax import lax
from jax.experimental import pallas as pl
from jax.experimental.pallas import tpu as pltpu
from jax.experimental.pallas import tpu_sc as plsc

N = 10000
E = 320000
L = 64
EPS = 1e-5

NC = 2
NS = 16
NW = NC * NS
CG = 128
NCH = E // CG
CW = 80
SBIG = 640
SLAST = N - 15 * SBIG
KBIG = SBIG // CW
KLAST = SLAST // CW

BE = 8000
GRID_E = E // BE

_F32 = jnp.float32


def _lnorm(h, g, b):
    mu = jnp.mean(h, axis=-1, keepdims=True)
    c = h - mu
    var = jnp.mean(c * c, axis=-1, keepdims=True)
    return c * lax.rsqrt(var + EPS) * g + b


def _dot(a, w):
    return jnp.dot(a, w, preferred_element_type=_F32)


def _prep_body(x_ref, w1, b1, w2, b2, lg, lb, wi, wj, h_ref, t_ref):
    h = jnp.maximum(_dot(x_ref[...], w1[...]) + b1[...], 0.0)
    h = _dot(h, w2[...]) + b2[...]
    h = _lnorm(h, lg[...], lb[...])
    h_ref[...] = h
    t_ref[...] = jnp.concatenate([_dot(h, wi[...]), _dot(h, wj[...])],
                                 axis=-1)


def _edge0_body(ef, ew1, eb1, ew2, eb2, elg, elb, g,
                we, sb1, sw2, sb2, slg, slb, m_ref, e1_ref):
    e = jnp.maximum(_dot(ef[...], ew1[...]) + eb1[...], 0.0)
    e = _dot(e, ew2[...]) + eb2[...]
    e = _lnorm(e, elg[...], elb[...])
    pre = g[...] + _dot(e, we[...]) + sb1[...]
    hid = jnp.maximum(pre, 0.0)
    m = _lnorm(_dot(hid, sw2[...]) + sb2[...], slg[...], slb[...])
    m_ref[...] = jnp.concatenate([m, jnp.zeros_like(m)], axis=-1)
    e1_ref[...] = m + e


def _edge1_body(e1, g, we, sb1, sw2, sb2, slg, slb, m_ref):
    pre = g[...] + _dot(e1[...], we[...]) + sb1[...]
    hid = jnp.maximum(pre, 0.0)
    m = _lnorm(_dot(hid, sw2[...]) + sb2[...], slg[...], slb[...])
    m_ref[...] = jnp.concatenate([m, jnp.zeros_like(m)], axis=-1)


def _node0_body(a0, a1, h_ref, wa, wh, b1, w2, b2, lg, lb, wi, wj,
                hn_ref, t_ref):
    h = h_ref[...]
    agg = a0[...] + a1[...]
    pre = _dot(agg, wa[...]) + _dot(h, wh[...]) + b1[...]
    hid = jnp.maximum(pre, 0.0)
    hn = _lnorm(_dot(hid, w2[...]) + b2[...], lg[...], lb[...]) + h
    hn_ref[...] = hn
    t_ref[...] = jnp.concatenate([_dot(hn, wi[...]), _dot(hn, wj[...])],
                                 axis=-1)


def _node1_body(a0, a1, h_ref, wa, wh, b1, w2, b2, lg, lb,
                dw1, db1, dw2, db2, out_ref):
    h = h_ref[...]
    agg = a0[...] + a1[...]
    pre = _dot(agg, wa[...]) + _dot(h, wh[...]) + b1[...]
    hid = jnp.maximum(pre, 0.0)
    hn = _lnorm(_dot(hid, w2[...]) + b2[...], lg[...], lb[...]) + h
    d = jnp.maximum(_dot(hn, dw1[...]) + db1[...], 0.0)
    out_ref[...] = _dot(d, dw2[...]) + db2[...]


def _full(shape):
    return pl.BlockSpec(shape, lambda i: (0, 0))


def _rows(shape):
    return pl.BlockSpec(shape, lambda i: (i, 0))


def _sc_gather_body(t12_hbm, ridx_hbm, sidx_hbm, g_hbm,
                    ridx_v, sidx_v, bufa, bufb, bufc, sema, semb):
    cid = lax.axis_index("c")
    sid = lax.axis_index("s")
    wid = sid * NC + cid
    nch = jnp.where(wid < NCH % NW, NCH // NW + 1, NCH // NW)

    def body(i, carry):
        ch = wid + i * NW
        pltpu.sync_copy(ridx_hbm.at[ch], ridx_v)
        pltpu.sync_copy(sidx_hbm.at[ch], sidx_v)
        ca = pltpu.async_copy(t12_hbm.at[ridx_v], bufa, sema)
        cb = pltpu.async_copy(t12_hbm.at[sidx_v], bufb, semb)
        ca.wait()
        cb.wait()

        def addrow(r, c2):
            for q in range(L // 16):
                a = bufa[r, pl.ds(q * 16, 16)]
                b = bufb[r, pl.ds(L + q * 16, 16)]
                bufc[r, pl.ds(q * 16, 16)] = a + b
            return c2

        lax.fori_loop(0, CG, addrow, 0)
        pltpu.sync_copy(bufc, g_hbm.at[pl.ds(ch * CG, CG)])
        return carry

    lax.fori_loop(0, nch, body, 0)


def _sc_scatter_body(m_hbm, ridx_hbm, zeros_hbm, out_hbm,
                     ridx_v, mbuf, stage_v, acc_sh, sem):
    cid = lax.axis_index("c")
    sid = lax.axis_index("s")
    wid = sid * NC + cid
    nch = jnp.where(wid < NCH % NW, NCH // NW + 1, NCH // NW)

    pltpu.sync_copy(zeros_hbm, stage_v)

    @pl.when(sid < 15)
    def _():
        for k in range(KBIG):
            pltpu.sync_copy(stage_v, acc_sh.at[pl.ds(sid * SBIG + k * CW, CW)])

    @pl.when(sid == 15)
    def _():
        for k in range(KLAST):
            pltpu.sync_copy(stage_v, acc_sh.at[pl.ds(15 * SBIG + k * CW, CW)])

    plsc.subcore_barrier()

    def body(i, carry):
        ch = wid + i * NW
        pltpu.sync_copy(ridx_hbm.at[ch], ridx_v)
        pltpu.sync_copy(m_hbm.at[pl.ds(ch * CG, CG)], mbuf)
        pltpu.sync_copy(mbuf, acc_sh.at[ridx_v], add=True)
        return carry

    lax.fori_loop(0, nch, body, 0)
    plsc.subcore_barrier()

    @pl.when(sid < 15)
    def _():
        for k in range(KBIG):
            off = sid * SBIG + k * CW
            pltpu.sync_copy(acc_sh.at[pl.ds(off, CW)], stage_v)
            pltpu.sync_copy(stage_v, out_hbm.at[pl.ds(cid * N + off, CW)])

    @pl.when(sid == 15)
    def _():
        for k in range(KLAST):
            off = 15 * SBIG + k * CW
            pltpu.sync_copy(acc_sh.at[pl.ds(off, CW)], stage_v)
            pltpu.sync_copy(stage_v, out_hbm.at[pl.ds(cid * N + off, CW)])


@functools.cache
def _sc_kernels():
    mesh = plsc.VectorSubcoreMesh(core_axis_name="c", subcore_axis_name="s")
    gather = pl.kernel(
        _sc_gather_body, mesh=mesh,
        out_type=jax.ShapeDtypeStruct((E, L), _F32),
        scratch_types=[
            pltpu.VMEM((CG,), jnp.int32),
            pltpu.VMEM((CG,), jnp.int32),
            pltpu.VMEM((CG, 2 * L), _F32),
            pltpu.VMEM((CG, 2 * L), _F32),
            pltpu.VMEM((CG, L), _F32),
            pltpu.SemaphoreType.DMA,
            pltpu.SemaphoreType.DMA,
        ])
    scatter = pl.kernel(
        _sc_scatter_body, mesh=mesh,
        out_type=jax.ShapeDtypeStruct((2 * N, 2 * L), _F32),
        scratch_types=[
            pltpu.VMEM((CG,), jnp.int32),
            pltpu.VMEM((CG, 2 * L), _F32),
            pltpu.VMEM((CW, 2 * L), _F32),
            pltpu.VMEM_SHARED((N, 2 * L), _F32),
            pltpu.SemaphoreType.DMA,
        ])
    return gather, scatter


def _vec(b):
    return b.reshape(1, -1)


def kernel(x, edge_index, e_features, params):
    recv = edge_index[1].astype(jnp.int32)
    send = edge_index[0].astype(jnp.int32)
    ridx2 = recv.reshape(NCH, CG)
    sidx2 = send.reshape(NCH, CG)
    zeros_cw = jnp.zeros((CW, 2 * L), _F32)

    (nw1, nb1), (nw2, nb2) = params["enc_node_mlp"]
    nlg, nlb = params["enc_node_ln"]
    (ew1, eb1), (ew2, eb2) = params["enc_edge_mlp"]
    elg, elb = params["enc_edge_ln"]
    (dw1, db1), (dw2, db2) = params["dec_mlp"]

    steps = []
    for sp in params["proc"]:
        (sW1, sb1), (sW2, sb2) = sp["edge_mlp"]
        slg, slb = sp["edge_ln"]
        (pW1, pb1), (pW2, pb2) = sp["node_mlp"]
        plg, plb = sp["node_ln"]
        steps.append(dict(
            Wi=sW1[:L], Wj=sW1[L:2 * L], We=sW1[2 * L:], b1=_vec(sb1),
            W2=sW2, b2=_vec(sb2), lg=_vec(slg), lb=_vec(slb),
            nWa=pW1[:L], nWh=pW1[L:], nb1=_vec(pb1),
            nW2=pW2, nb2=_vec(pb2), nlg=_vec(plg), nlb=_vec(plb)))

    s0, s1 = steps
    _sc_gather, _sc_scatter = _sc_kernels()

    h, t12 = pl.pallas_call(
        _prep_body,
        out_shape=(jax.ShapeDtypeStruct((N, L), _F32),
                   jax.ShapeDtypeStruct((N, 2 * L), _F32)),
    )(x, nw1, _vec(nb1), nw2, _vec(nb2), _vec(nlg), _vec(nlb),
      s0["Wi"], s0["Wj"])

    g = _sc_gather(t12, ridx2, sidx2)

    m0, e1 = pl.pallas_call(
        _edge0_body,
        grid=(GRID_E,),
        in_specs=[
            _rows((BE, 16)),
            _full((16, L)), _full((1, L)), _full((L, L)), _full((1, L)),
            _full((1, L)), _full((1, L)),
            _rows((BE, L)),
            _full((L, L)), _full((1, L)), _full((L, L)), _full((1, L)),
            _full((1, L)), _full((1, L)),
        ],
        out_specs=[_rows((BE, 2 * L)), _rows((BE, L))],
        out_shape=(jax.ShapeDtypeStruct((E, 2 * L), _F32),
                   jax.ShapeDtypeStruct((E, L), _F32)),
    )(e_features, ew1, _vec(eb1), ew2, _vec(eb2), _vec(elg), _vec(elb),
      g, s0["We"], s0["b1"], s0["W2"], s0["b2"], s0["lg"], s0["lb"])

    agg0 = _sc_scatter(m0, ridx2, zeros_cw)

    h, t12 = pl.pallas_call(
        _node0_body,
        out_shape=(jax.ShapeDtypeStruct((N, L), _F32),
                   jax.ShapeDtypeStruct((N, 2 * L), _F32)),
    )(agg0[:N, :L], agg0[N:, :L], h, s0["nWa"], s0["nWh"], s0["nb1"],
      s0["nW2"], s0["nb2"], s0["nlg"], s0["nlb"], s1["Wi"], s1["Wj"])

    g = _sc_gather(t12, ridx2, sidx2)

    m1 = pl.pallas_call(
        _edge1_body,
        grid=(GRID_E,),
        in_specs=[
            _rows((BE, L)), _rows((BE, L)),
            _full((L, L)), _full((1, L)), _full((L, L)), _full((1, L)),
            _full((1, L)), _full((1, L)),
        ],
        out_specs=_rows((BE, 2 * L)),
        out_shape=jax.ShapeDtypeStruct((E, 2 * L), _F32),
    )(e1, g, s1["We"], s1["b1"], s1["W2"], s1["b2"], s1["lg"], s1["lb"])

    agg1 = _sc_scatter(m1, ridx2, zeros_cw)

    out = pl.pallas_call(
        _node1_body,
        out_shape=jax.ShapeDtypeStruct((N, 3), _F32),
    )(agg1[:N, :L], agg1[N:, :L], h, s1["nWa"], s1["nWh"], s1["nb1"],
      s1["nW2"], s1["nb2"], s1["nlg"], s1["nlb"],
      dw1, _vec(db1), dw2, _vec(db2))

    return out


# speedup vs baseline: 3.1377x; 1.0503x over previous
import functools

import jax
import jax.numpy as jnp
from jax import lax
from jax.experimental import pallas as pl
from jax.experimental.pallas import tpu as pltpu
from jax.experimental.pallas import tpu_sc as plsc

N = 10000
E = 320000
L = 64
EPS = 1e-5

NC = 2
NS = 16
NW = NC * NS
CG = 128
NCH = E // CG
TPW = NCH // NW
TRIPS = TPW // 2
CW = 80
SBIG = 640
SLAST = N - 15 * SBIG
KBIG = SBIG // CW
KLAST = SLAST // CW

BE = 8000
GRID_E = E // BE

_F32 = jnp.float32


def _lnorm(h, g, b):
    mu = jnp.mean(h, axis=-1, keepdims=True)
    c = h - mu
    var = jnp.mean(c * c, axis=-1, keepdims=True)
    return c * lax.rsqrt(var + EPS) * g + b


def _dot(a, w):
    return jnp.dot(a, w, preferred_element_type=_F32)


def _prep_body(x_ref, w1, b1, w2, b2, lg, lb, wi, wj, h_ref, t_ref):
    h = jnp.maximum(_dot(x_ref[...], w1[...]) + b1[...], 0.0)
    h = _dot(h, w2[...]) + b2[...]
    h = _lnorm(h, lg[...], lb[...])
    h_ref[...] = h
    t_ref[...] = jnp.concatenate([_dot(h, wi[...]), _dot(h, wj[...])],
                                 axis=-1)


def _edge0_body(ef, ew1, eb1, ew2, eb2, elg, elb, g,
                we, sb1, sw2, sb2, slg, slb, m_ref, e1_ref):
    e = jnp.maximum(_dot(ef[...], ew1[...]) + eb1[...], 0.0)
    e = _dot(e, ew2[...]) + eb2[...]
    e = _lnorm(e, elg[...], elb[...])
    pre = g[...] + _dot(e, we[...]) + sb1[...]
    hid = jnp.maximum(pre, 0.0)
    m = _lnorm(_dot(hid, sw2[...]) + sb2[...], slg[...], slb[...])
    m_ref[...] = jnp.concatenate([m, jnp.zeros_like(m)], axis=-1)
    e1_ref[...] = m + e


def _edge1_body(e1, g, we, sb1, sw2, sb2, slg, slb, m_ref):
    pre = g[...] + _dot(e1[...], we[...]) + sb1[...]
    hid = jnp.maximum(pre, 0.0)
    m = _lnorm(_dot(hid, sw2[...]) + sb2[...], slg[...], slb[...])
    m_ref[...] = jnp.concatenate([m, jnp.zeros_like(m)], axis=-1)


def _node0_body(a0, a1, h_ref, wa, wh, b1, w2, b2, lg, lb, wi, wj,
                hn_ref, t_ref):
    h = h_ref[...]
    agg = a0[...] + a1[...]
    pre = _dot(agg, wa[...]) + _dot(h, wh[...]) + b1[...]
    hid = jnp.maximum(pre, 0.0)
    hn = _lnorm(_dot(hid, w2[...]) + b2[...], lg[...], lb[...]) + h
    hn_ref[...] = hn
    t_ref[...] = jnp.concatenate([_dot(hn, wi[...]), _dot(hn, wj[...])],
                                 axis=-1)


def _node1_body(a0, a1, h_ref, wa, wh, b1, w2, b2, lg, lb,
                dw1, db1, dw2, db2, out_ref):
    h = h_ref[...]
    agg = a0[...] + a1[...]
    pre = _dot(agg, wa[...]) + _dot(h, wh[...]) + b1[...]
    hid = jnp.maximum(pre, 0.0)
    hn = _lnorm(_dot(hid, w2[...]) + b2[...], lg[...], lb[...]) + h
    d = jnp.maximum(_dot(hn, dw1[...]) + db1[...], 0.0)
    out_ref[...] = _dot(d, dw2[...]) + db2[...]


def _full(shape):
    return pl.BlockSpec(shape, lambda i: (0, 0))


def _rows(shape):
    return pl.BlockSpec(shape, lambda i: (i, 0))


def _sc_gather_body(t12_hbm, ridx_hbm, sidx_hbm, g_hbm,
                    ridx_va, sidx_va, ridx_vb, sidx_vb,
                    bufa_a, bufb_a, bufc_a, bufa_b, bufb_b, bufc_b,
                    sema, semb, semoa, semob):
    cid = lax.axis_index("c")
    sid = lax.axis_index("s")
    wid = sid * NC + cid

    def add_half(ba, bb, bc):
        def addrow(r, c2):
            for q in range(L // 16):
                bc[r, pl.ds(q * 16, 16)] = (ba[r, pl.ds(q * 16, 16)]
                                            + bb[r, pl.ds(L + q * 16, 16)])
            return c2
        lax.fori_loop(0, CG, addrow, 0, unroll=8)

    def fire(idx_r, idx_s, ba, bb, sem):
        pltpu.async_copy(t12_hbm.at[idx_r], ba, sem)
        pltpu.async_copy(t12_hbm.at[idx_s], bb, sem)

    def drain_gather(ba, bb, sem):
        pltpu.make_async_copy(t12_hbm.at[pl.ds(0, CG)], ba, sem).wait()
        pltpu.make_async_copy(t12_hbm.at[pl.ds(0, CG)], bb, sem).wait()

    def drain_out(bc, sem):
        pltpu.make_async_copy(bc, g_hbm.at[pl.ds(0, CG)], sem).wait()

    pltpu.sync_copy(ridx_hbm.at[wid], ridx_va)
    pltpu.sync_copy(sidx_hbm.at[wid], sidx_va)
    fire(ridx_va, sidx_va, bufa_a, bufb_a, sema)

    def body(i, carry):
        c0 = wid + (2 * i) * NW
        c1 = c0 + NW
        pltpu.sync_copy(ridx_hbm.at[c1], ridx_vb)
        pltpu.sync_copy(sidx_hbm.at[c1], sidx_vb)
        drain_gather(bufa_a, bufb_a, sema)
        fire(ridx_vb, sidx_vb, bufa_b, bufb_b, semb)

        @pl.when(i > 0)
        def _():
            drain_out(bufc_a, semoa)

        add_half(bufa_a, bufb_a, bufc_a)
        pltpu.async_copy(bufc_a, g_hbm.at[pl.ds(c0 * CG, CG)], semoa)

        @pl.when(i + 1 < TRIPS)
        def _():
            pltpu.sync_copy(ridx_hbm.at[c0 + 2 * NW], ridx_va)
            pltpu.sync_copy(sidx_hbm.at[c0 + 2 * NW], sidx_va)
            fire(ridx_va, sidx_va, bufa_a, bufb_a, sema)

        drain_gather(bufa_b, bufb_b, semb)

        @pl.when(i > 0)
        def _():
            drain_out(bufc_b, semob)

        add_half(bufa_b, bufb_b, bufc_b)
        pltpu.async_copy(bufc_b, g_hbm.at[pl.ds(c1 * CG, CG)], semob)
        return carry

    lax.fori_loop(0, TRIPS, body, 0)
    drain_out(bufc_a, semoa)
    drain_out(bufc_b, semob)

    @pl.when(wid < NCH % NW)
    def _():
        ch = wid + TPW * NW
        pltpu.sync_copy(ridx_hbm.at[ch], ridx_va)
        pltpu.sync_copy(sidx_hbm.at[ch], sidx_va)
        fire(ridx_va, sidx_va, bufa_a, bufb_a, sema)
        drain_gather(bufa_a, bufb_a, sema)
        add_half(bufa_a, bufb_a, bufc_a)
        pltpu.sync_copy(bufc_a, g_hbm.at[pl.ds(ch * CG, CG)])


def _sc_scatter_body(m_hbm, ridx_hbm, zeros_hbm, out_hbm,
                     ridx_v, mbuf, stage_v, acc_sh, sem):
    cid = lax.axis_index("c")
    sid = lax.axis_index("s")
    wid = sid * NC + cid
    nch = jnp.where(wid < NCH % NW, NCH // NW + 1, NCH // NW)

    pltpu.sync_copy(zeros_hbm, stage_v)

    @pl.when(sid < 15)
    def _():
        for k in range(KBIG):
            pltpu.sync_copy(stage_v, acc_sh.at[pl.ds(sid * SBIG + k * CW, CW)])

    @pl.when(sid == 15)
    def _():
        for k in range(KLAST):
            pltpu.sync_copy(stage_v, acc_sh.at[pl.ds(15 * SBIG + k * CW, CW)])

    plsc.subcore_barrier()

    def body(i, carry):
        ch = wid + i * NW
        pltpu.sync_copy(ridx_hbm.at[ch], ridx_v)
        pltpu.sync_copy(m_hbm.at[pl.ds(ch * CG, CG)], mbuf)
        pltpu.sync_copy(mbuf, acc_sh.at[ridx_v], add=True)
        return carry

    lax.fori_loop(0, nch, body, 0)
    plsc.subcore_barrier()

    @pl.when(sid < 15)
    def _():
        for k in range(KBIG):
            off = sid * SBIG + k * CW
            pltpu.sync_copy(acc_sh.at[pl.ds(off, CW)], stage_v)
            pltpu.sync_copy(stage_v, out_hbm.at[pl.ds(cid * N + off, CW)])

    @pl.when(sid == 15)
    def _():
        for k in range(KLAST):
            off = 15 * SBIG + k * CW
            pltpu.sync_copy(acc_sh.at[pl.ds(off, CW)], stage_v)
            pltpu.sync_copy(stage_v, out_hbm.at[pl.ds(cid * N + off, CW)])


@functools.cache
def _sc_kernels():
    mesh = plsc.VectorSubcoreMesh(core_axis_name="c", subcore_axis_name="s")
    gather = pl.kernel(
        _sc_gather_body, mesh=mesh,
        out_type=jax.ShapeDtypeStruct((E, L), _F32),
        scratch_types=[
            pltpu.VMEM((CG,), jnp.int32),
            pltpu.VMEM((CG,), jnp.int32),
            pltpu.VMEM((CG,), jnp.int32),
            pltpu.VMEM((CG,), jnp.int32),
            pltpu.VMEM((CG, 2 * L), _F32),
            pltpu.VMEM((CG, 2 * L), _F32),
            pltpu.VMEM((CG, L), _F32),
            pltpu.VMEM((CG, 2 * L), _F32),
            pltpu.VMEM((CG, 2 * L), _F32),
            pltpu.VMEM((CG, L), _F32),
            pltpu.SemaphoreType.DMA,
            pltpu.SemaphoreType.DMA,
            pltpu.SemaphoreType.DMA,
            pltpu.SemaphoreType.DMA,
        ])
    scatter = pl.kernel(
        _sc_scatter_body, mesh=mesh,
        out_type=jax.ShapeDtypeStruct((2 * N, 2 * L), _F32),
        scratch_types=[
            pltpu.VMEM((CG,), jnp.int32),
            pltpu.VMEM((CG, 2 * L), _F32),
            pltpu.VMEM((CW, 2 * L), _F32),
            pltpu.VMEM_SHARED((N, 2 * L), _F32),
            pltpu.SemaphoreType.DMA,
        ])
    return gather, scatter


def _vec(b):
    return b.reshape(1, -1)


def kernel(x, edge_index, e_features, params):
    recv = edge_index[1].astype(jnp.int32)
    send = edge_index[0].astype(jnp.int32)
    ridx2 = recv.reshape(NCH, CG)
    sidx2 = send.reshape(NCH, CG)
    zeros_cw = jnp.zeros((CW, 2 * L), _F32)

    (nw1, nb1), (nw2, nb2) = params["enc_node_mlp"]
    nlg, nlb = params["enc_node_ln"]
    (ew1, eb1), (ew2, eb2) = params["enc_edge_mlp"]
    elg, elb = params["enc_edge_ln"]
    (dw1, db1), (dw2, db2) = params["dec_mlp"]

    steps = []
    for sp in params["proc"]:
        (sW1, sb1), (sW2, sb2) = sp["edge_mlp"]
        slg, slb = sp["edge_ln"]
        (pW1, pb1), (pW2, pb2) = sp["node_mlp"]
        plg, plb = sp["node_ln"]
        steps.append(dict(
            Wi=sW1[:L], Wj=sW1[L:2 * L], We=sW1[2 * L:], b1=_vec(sb1),
            W2=sW2, b2=_vec(sb2), lg=_vec(slg), lb=_vec(slb),
            nWa=pW1[:L], nWh=pW1[L:], nb1=_vec(pb1),
            nW2=pW2, nb2=_vec(pb2), nlg=_vec(plg), nlb=_vec(plb)))

    s0, s1 = steps
    _sc_gather, _sc_scatter = _sc_kernels()

    h, t12 = pl.pallas_call(
        _prep_body,
        out_shape=(jax.ShapeDtypeStruct((N, L), _F32),
                   jax.ShapeDtypeStruct((N, 2 * L), _F32)),
    )(x, nw1, _vec(nb1), nw2, _vec(nb2), _vec(nlg), _vec(nlb),
      s0["Wi"], s0["Wj"])

    g = _sc_gather(t12, ridx2, sidx2)

    m0, e1 = pl.pallas_call(
        _edge0_body,
        grid=(GRID_E,),
        in_specs=[
            _rows((BE, 16)),
            _full((16, L)), _full((1, L)), _full((L, L)), _full((1, L)),
            _full((1, L)), _full((1, L)),
            _rows((BE, L)),
            _full((L, L)), _full((1, L)), _full((L, L)), _full((1, L)),
            _full((1, L)), _full((1, L)),
        ],
        out_specs=[_rows((BE, 2 * L)), _rows((BE, L))],
        out_shape=(jax.ShapeDtypeStruct((E, 2 * L), _F32),
                   jax.ShapeDtypeStruct((E, L), _F32)),
    )(e_features, ew1, _vec(eb1), ew2, _vec(eb2), _vec(elg), _vec(elb),
      g, s0["We"], s0["b1"], s0["W2"], s0["b2"], s0["lg"], s0["lb"])

    agg0 = _sc_scatter(m0, ridx2, zeros_cw)

    h, t12 = pl.pallas_call(
        _node0_body,
        out_shape=(jax.ShapeDtypeStruct((N, L), _F32),
                   jax.ShapeDtypeStruct((N, 2 * L), _F32)),
    )(agg0[:N, :L], agg0[N:, :L], h, s0["nWa"], s0["nWh"], s0["nb1"],
      s0["nW2"], s0["nb2"], s0["nlg"], s0["nlb"], s1["Wi"], s1["Wj"])

    g = _sc_gather(t12, ridx2, sidx2)

    m1 = pl.pallas_call(
        _edge1_body,
        grid=(GRID_E,),
        in_specs=[
            _rows((BE, L)), _rows((BE, L)),
            _full((L, L)), _full((1, L)), _full((L, L)), _full((1, L)),
            _full((1, L)), _full((1, L)),
        ],
        out_specs=_rows((BE, 2 * L)),
        out_shape=jax.ShapeDtypeStruct((E, 2 * L), _F32),
    )(e1, g, s1["We"], s1["b1"], s1["W2"], s1["b2"], s1["lg"], s1["lb"])

    agg1 = _sc_scatter(m1, ridx2, zeros_cw)

    out = pl.pallas_call(
        _node1_body,
        out_shape=jax.ShapeDtypeStruct((N, 3), _F32),
    )(agg1[:N, :L], agg1[N:, :L], h, s1["nWa"], s1["nWh"], s1["nb1"],
      s1["nW2"], s1["nb2"], s1["nlg"], s1["nlb"],
      dw1, _vec(db1), dw2, _vec(db2))

    return out


# speedup vs baseline: 3.5337x; 1.1262x over previous
import functools

import jax
import jax.numpy as jnp
from jax import lax
from jax.experimental import pallas as pl
from jax.experimental.pallas import tpu as pltpu
from jax.experimental.pallas import tpu_sc as plsc

N = 10000
E = 320000
L = 64
EPS = 1e-5

NC = 2
NS = 16
NW = NC * NS
CG = 128
NCH = E // CG
TPW = NCH // NW
TRIPS = TPW // 2
CW = 80
SBIG = 640
SLAST = N - 15 * SBIG
KBIG = SBIG // CW
KLAST = SLAST // CW

BE = 8000
GRID_E = E // BE

_F32 = jnp.float32


def _lnorm(h, g, b):
    mu = jnp.mean(h, axis=-1, keepdims=True)
    c = h - mu
    var = jnp.mean(c * c, axis=-1, keepdims=True)
    return c * lax.rsqrt(var + EPS) * g + b


def _dot(a, w):
    return jnp.dot(a, w, preferred_element_type=_F32)


def _prep_body(x_ref, w1, b1, w2, b2, lg, lb, wi, wj, h_ref, t_ref):
    h = jnp.maximum(_dot(x_ref[...], w1[...]) + b1[...], 0.0)
    h = _dot(h, w2[...]) + b2[...]
    h = _lnorm(h, lg[...], lb[...])
    h_ref[...] = h
    t_ref[...] = jnp.concatenate([_dot(h, wi[...]), _dot(h, wj[...])],
                                 axis=-1)


def _edge0_body(ef, ew1, eb1, ew2, eb2, elg, elb, g,
                we, sb1, sw2, sb2, slg, slb, m_ref, e1_ref):
    e = jnp.maximum(_dot(ef[...], ew1[...]) + eb1[...], 0.0)
    e = _dot(e, ew2[...]) + eb2[...]
    e = _lnorm(e, elg[...], elb[...])
    pre = g[...] + _dot(e, we[...]) + sb1[...]
    hid = jnp.maximum(pre, 0.0)
    m = _lnorm(_dot(hid, sw2[...]) + sb2[...], slg[...], slb[...])
    m_ref[...] = jnp.concatenate([m, jnp.zeros_like(m)], axis=-1)
    e1_ref[...] = m + e


def _edge1_body(e1, g, we, sb1, sw2, sb2, slg, slb, m_ref):
    pre = g[...] + _dot(e1[...], we[...]) + sb1[...]
    hid = jnp.maximum(pre, 0.0)
    m = _lnorm(_dot(hid, sw2[...]) + sb2[...], slg[...], slb[...])
    m_ref[...] = jnp.concatenate([m, jnp.zeros_like(m)], axis=-1)


def _node0_body(a0, a1, h_ref, wa, wh, b1, w2, b2, lg, lb, wi, wj,
                hn_ref, t_ref):
    h = h_ref[...]
    agg = a0[...] + a1[...]
    pre = _dot(agg, wa[...]) + _dot(h, wh[...]) + b1[...]
    hid = jnp.maximum(pre, 0.0)
    hn = _lnorm(_dot(hid, w2[...]) + b2[...], lg[...], lb[...]) + h
    hn_ref[...] = hn
    t_ref[...] = jnp.concatenate([_dot(hn, wi[...]), _dot(hn, wj[...])],
                                 axis=-1)


def _node1_body(a0, a1, h_ref, wa, wh, b1, w2, b2, lg, lb,
                dw1, db1, dw2, db2, out_ref):
    h = h_ref[...]
    agg = a0[...] + a1[...]
    pre = _dot(agg, wa[...]) + _dot(h, wh[...]) + b1[...]
    hid = jnp.maximum(pre, 0.0)
    hn = _lnorm(_dot(hid, w2[...]) + b2[...], lg[...], lb[...]) + h
    d = jnp.maximum(_dot(hn, dw1[...]) + db1[...], 0.0)
    out_ref[...] = _dot(d, dw2[...]) + db2[...]


def _full(shape):
    return pl.BlockSpec(shape, lambda i: (0, 0))


def _rows(shape):
    return pl.BlockSpec(shape, lambda i: (i, 0))


def _sc_gather_body(t12_hbm, ridx_hbm, sidx_hbm, g_hbm,
                    ridx_va, sidx_va, ridx_vb, sidx_vb,
                    bufa_a, bufb_a, bufc_a, bufa_b, bufb_b, bufc_b,
                    sema, semb, semoa, semob):
    cid = lax.axis_index("c")
    sid = lax.axis_index("s")
    wid = sid * NC + cid


    def add_half(ba, bb, bc):
        def addrow(r, c2):
            for q in range(L // 16):
                bc[r, pl.ds(q * 16, 16)] = (ba[r, pl.ds(q * 16, 16)]
                                            + bb[r, pl.ds(L + q * 16, 16)])
            return c2
        lax.fori_loop(0, CG, addrow, 0, unroll=8)

    def fire(idx_r, idx_s, ba, bb, sem):
        pltpu.async_copy(t12_hbm.at[idx_r], ba, sem)
        pltpu.async_copy(t12_hbm.at[idx_s], bb, sem)

    def drain_gather(ba, bb, sem):
        pltpu.make_async_copy(t12_hbm.at[pl.ds(0, CG)], ba, sem).wait()
        pltpu.make_async_copy(t12_hbm.at[pl.ds(0, CG)], bb, sem).wait()

    def drain_out(bc, sem):
        pltpu.make_async_copy(bc, g_hbm.at[pl.ds(0, CG)], sem).wait()

    pltpu.sync_copy(ridx_hbm.at[wid], ridx_va)
    pltpu.sync_copy(sidx_hbm.at[wid], sidx_va)
    fire(ridx_va, sidx_va, bufa_a, bufb_a, sema)

    def body(i, carry):
        c0 = wid + (2 * i) * NW
        c1 = c0 + NW
        pltpu.sync_copy(ridx_hbm.at[c1], ridx_vb)
        pltpu.sync_copy(sidx_hbm.at[c1], sidx_vb)
        drain_gather(bufa_a, bufb_a, sema)
        fire(ridx_vb, sidx_vb, bufa_b, bufb_b, semb)

        @pl.when(i > 0)
        def _():
            drain_out(bufc_a, semoa)

        add_half(bufa_a, bufb_a, bufc_a)
        pltpu.async_copy(bufc_a, g_hbm.at[pl.ds(c0 * CG, CG)], semoa)

        @pl.when(i + 1 < TRIPS)
        def _():
            pltpu.sync_copy(ridx_hbm.at[c0 + 2 * NW], ridx_va)
            pltpu.sync_copy(sidx_hbm.at[c0 + 2 * NW], sidx_va)
            fire(ridx_va, sidx_va, bufa_a, bufb_a, sema)

        drain_gather(bufa_b, bufb_b, semb)

        @pl.when(i > 0)
        def _():
            drain_out(bufc_b, semob)

        add_half(bufa_b, bufb_b, bufc_b)
        pltpu.async_copy(bufc_b, g_hbm.at[pl.ds(c1 * CG, CG)], semob)
        return carry

    lax.fori_loop(0, TRIPS, body, 0)
    drain_out(bufc_a, semoa)
    drain_out(bufc_b, semob)

    @pl.when(wid < NCH % NW)
    def _():
        ch = wid + TPW * NW
        pltpu.sync_copy(ridx_hbm.at[ch], ridx_va)
        pltpu.sync_copy(sidx_hbm.at[ch], sidx_va)
        fire(ridx_va, sidx_va, bufa_a, bufb_a, sema)
        drain_gather(bufa_a, bufb_a, sema)
        add_half(bufa_a, bufb_a, bufc_a)
        pltpu.sync_copy(bufc_a, g_hbm.at[pl.ds(ch * CG, CG)])


def _sc_scatter_body(m_hbm, ridx_hbm, zeros_hbm, out_hbm,
                     ridx_va, ridx_vb, mbuf_a, mbuf_b, stage_v, acc_sh,
                     semma, semmb):
    cid = lax.axis_index("c")
    sid = lax.axis_index("s")
    wid = sid * NC + cid

    pltpu.sync_copy(zeros_hbm, stage_v)

    @pl.when(sid < 15)
    def _():
        for k in range(KBIG):
            pltpu.sync_copy(stage_v, acc_sh.at[pl.ds(sid * SBIG + k * CW, CW)])

    @pl.when(sid == 15)
    def _():
        for k in range(KLAST):
            pltpu.sync_copy(stage_v, acc_sh.at[pl.ds(15 * SBIG + k * CW, CW)])

    plsc.subcore_barrier()

    def drain(buf, sem):
        pltpu.make_async_copy(m_hbm.at[pl.ds(0, CG)], buf, sem).wait()

    pltpu.sync_copy(ridx_hbm.at[wid], ridx_va)
    pltpu.async_copy(m_hbm.at[pl.ds(wid * CG, CG)], mbuf_a, semma)

    def body(i, carry):
        c0 = wid + (2 * i) * NW
        c1 = c0 + NW
        pltpu.sync_copy(ridx_hbm.at[c1], ridx_vb)
        pltpu.async_copy(m_hbm.at[pl.ds(c1 * CG, CG)], mbuf_b, semmb)
        drain(mbuf_a, semma)
        pltpu.sync_copy(mbuf_a, acc_sh.at[ridx_va], add=True)

        @pl.when(i + 1 < TRIPS)
        def _():
            pltpu.sync_copy(ridx_hbm.at[c0 + 2 * NW], ridx_va)
            pltpu.async_copy(m_hbm.at[pl.ds((c0 + 2 * NW) * CG, CG)],
                             mbuf_a, semma)

        drain(mbuf_b, semmb)
        pltpu.sync_copy(mbuf_b, acc_sh.at[ridx_vb], add=True)
        return carry

    lax.fori_loop(0, TRIPS, body, 0)

    @pl.when(wid < NCH % NW)
    def _():
        ch = wid + TPW * NW
        pltpu.sync_copy(ridx_hbm.at[ch], ridx_va)
        pltpu.sync_copy(m_hbm.at[pl.ds(ch * CG, CG)], mbuf_a)
        pltpu.sync_copy(mbuf_a, acc_sh.at[ridx_va], add=True)

    plsc.subcore_barrier()

    @pl.when(sid < 15)
    def _():
        for k in range(KBIG):
            off = sid * SBIG + k * CW
            pltpu.sync_copy(acc_sh.at[pl.ds(off, CW)], stage_v)
            pltpu.sync_copy(stage_v, out_hbm.at[pl.ds(cid * N + off, CW)])

    @pl.when(sid == 15)
    def _():
        for k in range(KLAST):
            off = 15 * SBIG + k * CW
            pltpu.sync_copy(acc_sh.at[pl.ds(off, CW)], stage_v)
            pltpu.sync_copy(stage_v, out_hbm.at[pl.ds(cid * N + off, CW)])


@functools.cache
def _sc_kernels():
    mesh = plsc.VectorSubcoreMesh(core_axis_name="c", subcore_axis_name="s")
    gather = pl.kernel(
        _sc_gather_body, mesh=mesh,
        out_type=jax.ShapeDtypeStruct((E, L), _F32),
        scratch_types=[
            pltpu.VMEM((CG,), jnp.int32),
            pltpu.VMEM((CG,), jnp.int32),
            pltpu.VMEM((CG,), jnp.int32),
            pltpu.VMEM((CG,), jnp.int32),
            pltpu.VMEM((CG, 2 * L), _F32),
            pltpu.VMEM((CG, 2 * L), _F32),
            pltpu.VMEM((CG, L), _F32),
            pltpu.VMEM((CG, 2 * L), _F32),
            pltpu.VMEM((CG, 2 * L), _F32),
            pltpu.VMEM((CG, L), _F32),
            pltpu.SemaphoreType.DMA,
            pltpu.SemaphoreType.DMA,
            pltpu.SemaphoreType.DMA,
            pltpu.SemaphoreType.DMA,
        ])
    scatter = pl.kernel(
        _sc_scatter_body, mesh=mesh,
        out_type=jax.ShapeDtypeStruct((2 * N, 2 * L), _F32),
        scratch_types=[
            pltpu.VMEM((CG,), jnp.int32),
            pltpu.VMEM((CG,), jnp.int32),
            pltpu.VMEM((CG, 2 * L), _F32),
            pltpu.VMEM((CG, 2 * L), _F32),
            pltpu.VMEM((CW, 2 * L), _F32),
            pltpu.VMEM_SHARED((N, 2 * L), _F32),
            pltpu.SemaphoreType.DMA,
            pltpu.SemaphoreType.DMA,
        ])
    return gather, scatter


def _vec(b):
    return b.reshape(1, -1)


def kernel(x, edge_index, e_features, params):
    recv = edge_index[1].astype(jnp.int32)
    send = edge_index[0].astype(jnp.int32)
    ridx2 = recv.reshape(NCH, CG)
    sidx2 = send.reshape(NCH, CG)
    zeros_cw = jnp.zeros((CW, 2 * L), _F32)

    (nw1, nb1), (nw2, nb2) = params["enc_node_mlp"]
    nlg, nlb = params["enc_node_ln"]
    (ew1, eb1), (ew2, eb2) = params["enc_edge_mlp"]
    elg, elb = params["enc_edge_ln"]
    (dw1, db1), (dw2, db2) = params["dec_mlp"]

    steps = []
    for sp in params["proc"]:
        (sW1, sb1), (sW2, sb2) = sp["edge_mlp"]
        slg, slb = sp["edge_ln"]
        (pW1, pb1), (pW2, pb2) = sp["node_mlp"]
        plg, plb = sp["node_ln"]
        steps.append(dict(
            Wi=sW1[:L], Wj=sW1[L:2 * L], We=sW1[2 * L:], b1=_vec(sb1),
            W2=sW2, b2=_vec(sb2), lg=_vec(slg), lb=_vec(slb),
            nWa=pW1[:L], nWh=pW1[L:], nb1=_vec(pb1),
            nW2=pW2, nb2=_vec(pb2), nlg=_vec(plg), nlb=_vec(plb)))

    s0, s1 = steps
    _sc_gather, _sc_scatter = _sc_kernels()

    h, t12 = pl.pallas_call(
        _prep_body,
        out_shape=(jax.ShapeDtypeStruct((N, L), _F32),
                   jax.ShapeDtypeStruct((N, 2 * L), _F32)),
    )(x, nw1, _vec(nb1), nw2, _vec(nb2), _vec(nlg), _vec(nlb),
      s0["Wi"], s0["Wj"])

    g = _sc_gather(t12, ridx2, sidx2)

    m0, e1 = pl.pallas_call(
        _edge0_body,
        grid=(GRID_E,),
        in_specs=[
            _rows((BE, 16)),
            _full((16, L)), _full((1, L)), _full((L, L)), _full((1, L)),
            _full((1, L)), _full((1, L)),
            _rows((BE, L)),
            _full((L, L)), _full((1, L)), _full((L, L)), _full((1, L)),
            _full((1, L)), _full((1, L)),
        ],
        out_specs=[_rows((BE, 2 * L)), _rows((BE, L))],
        out_shape=(jax.ShapeDtypeStruct((E, 2 * L), _F32),
                   jax.ShapeDtypeStruct((E, L), _F32)),
    )(e_features, ew1, _vec(eb1), ew2, _vec(eb2), _vec(elg), _vec(elb),
      g, s0["We"], s0["b1"], s0["W2"], s0["b2"], s0["lg"], s0["lb"])

    agg0 = _sc_scatter(m0, ridx2, zeros_cw)

    h, t12 = pl.pallas_call(
        _node0_body,
        out_shape=(jax.ShapeDtypeStruct((N, L), _F32),
                   jax.ShapeDtypeStruct((N, 2 * L), _F32)),
    )(agg0[:N, :L], agg0[N:, :L], h, s0["nWa"], s0["nWh"], s0["nb1"],
      s0["nW2"], s0["nb2"], s0["nlg"], s0["nlb"], s1["Wi"], s1["Wj"])

    g = _sc_gather(t12, ridx2, sidx2)

    m1 = pl.pallas_call(
        _edge1_body,
        grid=(GRID_E,),
        in_specs=[
            _rows((BE, L)), _rows((BE, L)),
            _full((L, L)), _full((1, L)), _full((L, L)), _full((1, L)),
            _full((1, L)), _full((1, L)),
        ],
        out_specs=_rows((BE, 2 * L)),
        out_shape=jax.ShapeDtypeStruct((E, 2 * L), _F32),
    )(e1, g, s1["We"], s1["b1"], s1["W2"], s1["b2"], s1["lg"], s1["lb"])

    agg1 = _sc_scatter(m1, ridx2, zeros_cw)

    out = pl.pallas_call(
        _node1_body,
        out_shape=jax.ShapeDtypeStruct((N, 3), _F32),
    )(agg1[:N, :L], agg1[N:, :L], h, s1["nWa"], s1["nWh"], s1["nb1"],
      s1["nW2"], s1["nb2"], s1["nlg"], s1["nlb"],
      dw1, _vec(db1), dw2, _vec(db2))

    return out
